# Initial kernel scaffold; baseline (speedup 1.0000x reference)
import jax
import jax.numpy as jnp
from jax.experimental import pallas as pl


def kernel(atomic_numbers, neighbour_indices, neighbour_displacements, embed_table, W_rad, W_td, Wq1, Wk1, Wv1, Wo1, Wb1, Wq2, Wk2, Wv2, Wo2, Wb2, W_embed, b_embed):
    raise NotImplementedError("write your pallas kernel here")


# speedup vs baseline: 5.0192x; 5.0192x over previous
import functools
import numpy as np
import jax
import jax.numpy as jnp
from jax import lax
from jax.experimental import pallas as pl
from jax.experimental.pallas import tpu as pltpu
from jax.experimental.pallas import tpu_sc as plsc

N = 10000
E = 160000
F = 64
NR = 16
CUT = 5.0
DEG = (0, 1, 1, 1, 2, 2, 2, 2, 2)
SCALE = 1.0 / np.sqrt(9 * F)
EB = 3200
NB = 2000
NEG = -1e30


def _k1_body(disp_ref, zj_ref, et_ref, wrad_ref, wtd_ref, wb1_ref, wb2_ref,
             bda1_ref, rec_ref, sh_ref, l1_ref, mg_ref):
    pid = pl.program_id(0)
    d = disp_ref[...]
    x = d[:, 0:1]; y = d[:, 1:2]; z = d[:, 2:3]
    r2 = x * x + y * y + z * z
    r = jnp.sqrt(r2 + 1e-12)
    ux = x / r; uy = y / r; uz = z / r
    k = lax.broadcasted_iota(jnp.int32, (EB, NR), 1).astype(jnp.float32) + 1.0
    arg = jnp.pi * (k * r / CUT)
    rb0 = jnp.sin(arg) / arg
    env = jnp.where(r < CUT, 0.5 * (jnp.cos(jnp.pi * r / CUT) + 1.0), 0.0)
    rb = rb0 * env
    c0 = 0.28209479177387814
    c1c = 0.4886025119029199
    c2c = 1.0925484305920792
    sh_cols = [
        jnp.full((EB, 1), c0, jnp.float32),
        c1c * uy, c1c * uz, c1c * ux,
        c2c * ux * uy, c2c * uy * uz,
        0.31539156525252005 * (3.0 * uz * uz - 1.0),
        c2c * ux * uz,
        0.5462742152960396 * (ux * ux - uy * uy),
        jnp.zeros((EB, 7), jnp.float32),
    ]
    sh = jnp.concatenate(sh_cols, axis=1)
    zj = zj_ref[...]
    io = lax.broadcasted_iota(jnp.int32, (EB, 128), 1)
    oh = (zj == io).astype(jnp.float32)
    e_j = jnp.dot(oh, et_ref[...], preferred_element_type=jnp.float32)
    y0 = jnp.dot(rb, wrad_ref[...], preferred_element_type=jnp.float32) * e_j
    t = jnp.dot(y0, wtd_ref[...], preferred_element_type=jnp.float32)
    c1 = jnp.dot(rb, wb1_ref[...], preferred_element_type=jnp.float32)
    c2 = jnp.dot(rb, wb2_ref[...], preferred_element_type=jnp.float32)
    tc1 = t * jnp.concatenate([c1, c1, c1], axis=1)
    p = jnp.dot(tc1, bda1_ref[...], preferred_element_type=jnp.float32)
    sp = t * p
    s0 = jnp.sum(sp[:, 0:64], axis=1, keepdims=True)
    s1 = jnp.sum(sp[:, 64:128], axis=1, keepdims=True)
    s2 = jnp.sum(sp[:, 128:192], axis=1, keepdims=True)
    sh3 = sh * sh * sh
    S30 = sh3[:, 0:1]
    S31 = jnp.sum(sh3[:, 1:4], axis=1, keepdims=True)
    S32 = jnp.sum(sh3[:, 4:9], axis=1, keepdims=True)
    l1 = SCALE * (S30 * s0 + S31 * s1 + S32 * s2)
    rec_ref[...] = jnp.concatenate([t, c2], axis=1)
    sh_ref[...] = sh
    l1_ref[...] = jnp.broadcast_to(l1, (EB, 8))
    bmax = jnp.max(l1)

    @pl.when(pid == 0)
    def _():
        mg_ref[...] = jnp.full((8, 128), NEG, jnp.float32)

    mg_ref[...] = jnp.maximum(mg_ref[...], jnp.full((8, 128), bmax))


def _k1(disp_p, zj, etp, wrad, wtd, wb1, wb2, bda1):
    grid = (E // EB,)
    return pl.pallas_call(
        _k1_body,
        grid=grid,
        in_specs=[
            pl.BlockSpec((EB, 8), lambda i: (i, 0)),
            pl.BlockSpec((EB, 1), lambda i: (i, 0)),
            pl.BlockSpec((128, 64), lambda i: (0, 0)),
            pl.BlockSpec((NR, 64), lambda i: (0, 0)),
            pl.BlockSpec((64, 192), lambda i: (0, 0)),
            pl.BlockSpec((NR, 64), lambda i: (0, 0)),
            pl.BlockSpec((NR, 64), lambda i: (0, 0)),
            pl.BlockSpec((192, 192), lambda i: (0, 0)),
        ],
        out_specs=[
            pl.BlockSpec((EB, 256), lambda i: (i, 0)),
            pl.BlockSpec((EB, 16), lambda i: (i, 0)),
            pl.BlockSpec((EB, 8), lambda i: (i, 0)),
            pl.BlockSpec((8, 128), lambda i: (0, 0)),
        ],
        out_shape=[
            jax.ShapeDtypeStruct((E, 256), jnp.float32),
            jax.ShapeDtypeStruct((E, 16), jnp.float32),
            jax.ShapeDtypeStruct((E, 8), jnp.float32),
            jax.ShapeDtypeStruct((8, 128), jnp.float32),
        ],
    )(disp_p, zj, etp, wrad, wtd, wb1, wb2, bda1)


def _k4_body(att_ref, sh_ref, rec_ref, p1_ref):
    w = att_ref[:, 0:1] * sh_ref[...]
    rec = rec_ref[...]
    cols = [w[:, n:n + 1] * rec[:, DEG[n] * 64:(DEG[n] + 1) * 64] for n in range(9)]
    p1_ref[...] = jnp.concatenate(cols, axis=1)


def _k4(att8, sh, rec):
    return pl.pallas_call(
        _k4_body,
        grid=(E // EB,),
        in_specs=[
            pl.BlockSpec((EB, 8), lambda i: (i, 0)),
            pl.BlockSpec((EB, 16), lambda i: (i, 0)),
            pl.BlockSpec((EB, 256), lambda i: (i, 0)),
        ],
        out_specs=pl.BlockSpec((EB, 576), lambda i: (i, 0)),
        out_shape=jax.ShapeDtypeStruct((E, 576), jnp.float32),
    )(att8, sh, rec)


def _k6_body(msg_ref, bd1_ref, bda2_ref, h1_ref, ha_ref):
    h = jnp.dot(msg_ref[...], bd1_ref[...], preferred_element_type=jnp.float32)
    h1_ref[...] = h
    ha_ref[...] = jnp.dot(h, bda2_ref[...], preferred_element_type=jnp.float32)


def _k6(msg1, bd1, bda2):
    return pl.pallas_call(
        _k6_body,
        grid=(N // NB,),
        in_specs=[
            pl.BlockSpec((NB, 576), lambda i: (i, 0)),
            pl.BlockSpec((576, 576), lambda i: (0, 0)),
            pl.BlockSpec((576, 576), lambda i: (0, 0)),
        ],
        out_specs=[
            pl.BlockSpec((NB, 576), lambda i: (i, 0)),
            pl.BlockSpec((NB, 576), lambda i: (i, 0)),
        ],
        out_shape=[
            jax.ShapeDtypeStruct((N, 576), jnp.float32),
            jax.ShapeDtypeStruct((N, 576), jnp.float32),
        ],
    )(msg1, bd1, bda2)


def _k8_body(gi_ref, gj_ref, rec_ref, sh_ref, l2_ref, mg_ref):
    pid = pl.program_id(0)
    zz = gi_ref[...] * gj_ref[...]
    c2 = rec_ref[:, 192:256]
    sh = sh_ref[...]
    acc = jnp.zeros((EB, 1), jnp.float32)
    for n in range(9):
        sn = jnp.sum(zz[:, n * 64:(n + 1) * 64] * c2, axis=1, keepdims=True)
        acc = acc + sh[:, n:n + 1] * sn
    l2 = SCALE * acc
    l2_ref[...] = jnp.broadcast_to(l2, (EB, 8))
    bmax = jnp.max(l2)

    @pl.when(pid == 0)
    def _():
        mg_ref[...] = jnp.full((8, 128), NEG, jnp.float32)

    mg_ref[...] = jnp.maximum(mg_ref[...], jnp.full((8, 128), bmax))


def _k8(gi, gj, rec, sh):
    return pl.pallas_call(
        _k8_body,
        grid=(E // EB,),
        in_specs=[
            pl.BlockSpec((EB, 576), lambda i: (i, 0)),
            pl.BlockSpec((EB, 576), lambda i: (i, 0)),
            pl.BlockSpec((EB, 256), lambda i: (i, 0)),
            pl.BlockSpec((EB, 16), lambda i: (i, 0)),
        ],
        out_specs=[
            pl.BlockSpec((EB, 8), lambda i: (i, 0)),
            pl.BlockSpec((8, 128), lambda i: (0, 0)),
        ],
        out_shape=[
            jax.ShapeDtypeStruct((E, 8), jnp.float32),
            jax.ShapeDtypeStruct((8, 128), jnp.float32),
        ],
    )(gi, gj, rec, sh)


def _k10_body(att_ref, gj_ref, p2_ref):
    p2_ref[...] = att_ref[:, 0:1] * gj_ref[...]


def _k10(att8, gj):
    return pl.pallas_call(
        _k10_body,
        grid=(E // EB,),
        in_specs=[
            pl.BlockSpec((EB, 8), lambda i: (i, 0)),
            pl.BlockSpec((EB, 576), lambda i: (i, 0)),
        ],
        out_specs=pl.BlockSpec((EB, 576), lambda i: (i, 0)),
        out_shape=jax.ShapeDtypeStruct((E, 576), jnp.float32),
    )(att8, gj)


def _k12_body(h1_ref, msg_ref, bd2_ref, an_ref, et_ref, we_ref, be_ref, out_ref):
    out = h1_ref[...] + jnp.dot(msg_ref[...], bd2_ref[...], preferred_element_type=jnp.float32)
    an = an_ref[...]
    io = lax.broadcasted_iota(jnp.int32, (NB, 128), 1)
    oh = (an == io).astype(jnp.float32)
    en = jnp.dot(oh, et_ref[...], preferred_element_type=jnp.float32)
    emb = jnp.dot(en, we_ref[...], preferred_element_type=jnp.float32) + be_ref[0:1, :]
    out_ref[...] = jnp.concatenate([out[:, 0:64] + emb, out[:, 64:576]], axis=1)


def _k12(h1, msg2, bd2, an2, etp, we, be8):
    return pl.pallas_call(
        _k12_body,
        grid=(N // NB,),
        in_specs=[
            pl.BlockSpec((NB, 576), lambda i: (i, 0)),
            pl.BlockSpec((NB, 576), lambda i: (i, 0)),
            pl.BlockSpec((576, 576), lambda i: (0, 0)),
            pl.BlockSpec((NB, 1), lambda i: (i, 0)),
            pl.BlockSpec((128, 64), lambda i: (0, 0)),
            pl.BlockSpec((64, 64), lambda i: (0, 0)),
            pl.BlockSpec((8, 64), lambda i: (0, 0)),
        ],
        out_specs=pl.BlockSpec((NB, 576), lambda i: (i, 0)),
        out_shape=jax.ShapeDtypeStruct((N, 576), jnp.float32),
    )(h1, msg2, bd2, an2, etp, we, be8)


def kernel(atomic_numbers, neighbour_indices, neighbour_displacements,
           embed_table, W_rad, W_td,
           Wq1, Wk1, Wv1, Wo1, Wb1,
           Wq2, Wk2, Wv2, Wo2, Wb2,
           W_embed, b_embed):
    an = atomic_numbers.astype(jnp.int32)
    idx_i = neighbour_indices[:, 0].astype(jnp.int32)
    idx_j = neighbour_indices[:, 1].astype(jnp.int32)
    disp_p = jnp.pad(neighbour_displacements, ((0, 0), (0, 5)))
    etp = jnp.pad(embed_table, ((0, 28), (0, 0)))
    wtd = jnp.transpose(W_td, (1, 0, 2)).reshape(64, 192)
    A1 = Wq1 @ Wk1.T
    A2 = Wq2 @ Wk2.T
    bda1 = jnp.kron(jnp.eye(3, dtype=jnp.float32), A1.T)
    bd1 = jnp.kron(jnp.eye(9, dtype=jnp.float32), Wv1 @ Wo1)
    bda2 = jnp.kron(jnp.eye(9, dtype=jnp.float32), A2)
    bd2 = jnp.kron(jnp.eye(9, dtype=jnp.float32), Wv2 @ Wo2)
    be8 = jnp.broadcast_to(b_embed[None, :], (8, 64))
    an2 = an[:, None]

    zj = an[idx_j][:, None]

    rec, sh, l18, mg_t = _k1(disp_p, zj, etp, W_rad, wtd, Wb1, Wb2, bda1)
    mg = mg_t[0, 0]
    l1 = l18[:, 0]

    ex = jnp.exp(l1 - mg)
    s1 = jax.ops.segment_sum(ex, idx_i, num_segments=N)
    att1 = ex / (s1[idx_i] + 1e-9)
    att18 = jnp.broadcast_to(att1[:, None], (E, 8))

    p1 = _k4(att18, sh, rec)

    msg1 = jax.ops.segment_sum(p1, idx_i, num_segments=N)

    h1, ha = _k6(msg1, bd1, bda2)

    gi = ha[idx_i]
    gj = h1[idx_j]

    l28, mg2_t = _k8(gi, gj, rec, sh)
    mg2 = mg2_t[0, 0]
    l2 = l28[:, 0]

    ex2 = jnp.exp(l2 - mg2)
    s2 = jax.ops.segment_sum(ex2, idx_i, num_segments=N)
    att2 = ex2 / (s2[idx_i] + 1e-9)
    att28 = jnp.broadcast_to(att2[:, None], (E, 8))

    p2 = _k10(att28, gj)

    msg2 = jax.ops.segment_sum(p2, idx_i, num_segments=N)

    out = _k12(h1, msg2, bd2, an2, etp, W_embed, be8)
    return out.reshape(N, 9, F)


# speedup vs baseline: 6.3805x; 1.2712x over previous
import functools
import numpy as np
import jax
import jax.numpy as jnp
from jax import lax
from jax.experimental import pallas as pl
from jax.experimental.pallas import tpu as pltpu
from jax.experimental.pallas import tpu_sc as plsc

N = 10000
E = 160000
F = 64
NR = 16
CUT = 5.0
DEG = (0, 1, 1, 1, 2, 2, 2, 2, 2)
SCALE = 1.0 / np.sqrt(9 * F)
EB = 3200
NB = 1000
NEG = -1e30

NC = 2
NS = 16
NW = NC * NS
SPAN = E // NW
CH = 1000
NV = CH // 16 + 1
RB = 40

_MESH = plsc.VectorSubcoreMesh(core_axis_name="c", subcore_axis_name="s",
                               num_cores=NC, num_subcores=NS)


def _wid():
    return lax.axis_index("s") * NC + lax.axis_index("c")


CHK = 128
NCHUNK = E // CHK


def _nck(w):
    return jnp.where(w < NCHUNK - 39 * NW, 40, 39)


@functools.partial(
    pl.kernel,
    compiler_params=pltpu.CompilerParams(use_tc_tiling_on_sc=False),
    out_type=jax.ShapeDtypeStruct((E,), jnp.int32),
    mesh=_MESH,
    scratch_types=[
        pltpu.VMEM((CHK,), jnp.int32),
        pltpu.VMEM((CHK,), jnp.int32),
    ],
)
def _sc_zj(an_hbm, idxj_hbm, out_hbm, idx_v, out_v):
    w = _wid()

    def chunk(k, _):
        off = pl.multiple_of((w + NW * k) * CHK, CHK)
        pltpu.sync_copy(idxj_hbm.at[pl.ds(off, CHK)], idx_v)
        pltpu.sync_copy(an_hbm.at[idx_v], out_v)
        pltpu.sync_copy(out_v, out_hbm.at[pl.ds(off, CHK)])
        return 0

    lax.fori_loop(0, _nck(w), chunk, 0)


@functools.partial(
    pl.kernel,
    compiler_params=pltpu.CompilerParams(use_tc_tiling_on_sc=False),
    out_type=jax.ShapeDtypeStruct((NC, N), jnp.float32),
    mesh=_MESH,
    scratch_types=[
        pltpu.VMEM_SHARED((N,), jnp.float32),
        pltpu.VMEM((1024,), jnp.float32),
        pltpu.VMEM((CHK,), jnp.float32),
        pltpu.VMEM((CHK,), jnp.int32),
        pltpu.VMEM((CHK,), jnp.float32),
        pltpu.VMEM((16,), jnp.float32),
    ],
)
def _sc_ssum(l_hbm, idx_hbm, mg_hbm, out_hbm, acc, zb, l_v, idx_v, ex_v, mg_v):
    cc = lax.axis_index("c")
    sid = lax.axis_index("s")
    w = _wid()
    pltpu.sync_copy(mg_hbm, mg_v)

    @pl.when(sid == 0)
    def _():
        def zv(j, _):
            zb[pl.ds(j * 16, 16)] = jnp.zeros((16,), jnp.float32)
            return 0
        lax.fori_loop(0, 64, zv, 0)

        def zc(k, _):
            pltpu.sync_copy(zb, acc.at[pl.ds(pl.multiple_of(k * 1024, 1024), 1024)])
            return 0
        lax.fori_loop(0, N // 1024, zc, 0)
        pltpu.sync_copy(zb.at[pl.ds(0, N - 1024 * (N // 1024))],
                        acc.at[pl.ds(1024 * (N // 1024), N - 1024 * (N // 1024))])

    plsc.subcore_barrier()
    mg = mg_v[...]

    def chunk(k, _):
        off = pl.multiple_of((w + NW * k) * CHK, CHK)
        pltpu.sync_copy(l_hbm.at[pl.ds(off, CHK)], l_v)
        pltpu.sync_copy(idx_hbm.at[pl.ds(off, CHK)], idx_v)

        def vloop(j, _):
            ex_v[pl.ds(j * 16, 16)] = jnp.exp(l_v[pl.ds(j * 16, 16)] - mg)
            return 0

        lax.fori_loop(0, CHK // 16, vloop, 0)
        pltpu.sync_copy(ex_v, acc.at[idx_v], add=True)
        return 0

    lax.fori_loop(0, _nck(w), chunk, 0)
    plsc.subcore_barrier()

    @pl.when(sid == 0)
    def _():
        pltpu.sync_copy(acc, out_hbm.at[cc])


@functools.partial(
    pl.kernel,
    compiler_params=pltpu.CompilerParams(use_tc_tiling_on_sc=False),
    out_type=jax.ShapeDtypeStruct((E,), jnp.float32),
    mesh=_MESH,
    scratch_types=[
        pltpu.VMEM((CHK,), jnp.float32),
        pltpu.VMEM((CHK,), jnp.int32),
        pltpu.VMEM((CHK,), jnp.float32),
        pltpu.VMEM((CHK,), jnp.float32),
        pltpu.VMEM((CHK,), jnp.float32),
        pltpu.VMEM((16,), jnp.float32),
    ],
)
def _sc_att(s0_hbm, s1_hbm, l_hbm, idx_hbm, mg_hbm, out_hbm,
            l_v, idx_v, s0_v, s1_v, o_v, mg_v):
    w = _wid()
    pltpu.sync_copy(mg_hbm, mg_v)
    mg = mg_v[...]

    def chunk(k, _):
        off = pl.multiple_of((w + NW * k) * CHK, CHK)
        pltpu.sync_copy(l_hbm.at[pl.ds(off, CHK)], l_v)
        pltpu.sync_copy(idx_hbm.at[pl.ds(off, CHK)], idx_v)
        pltpu.sync_copy(s0_hbm.at[idx_v], s0_v)
        pltpu.sync_copy(s1_hbm.at[idx_v], s1_v)

        def vloop(j, _):
            sl = pl.ds(j * 16, 16)
            ex = jnp.exp(l_v[sl] - mg)
            o_v[sl] = ex / (s0_v[sl] + s1_v[sl] + 1e-9)
            return 0

        lax.fori_loop(0, CHK // 16, vloop, 0)
        pltpu.sync_copy(o_v, out_hbm.at[pl.ds(off, CHK)])
        return 0

    lax.fori_loop(0, _nck(w), chunk, 0)


@functools.partial(
    pl.kernel,
    compiler_params=pltpu.CompilerParams(use_tc_tiling_on_sc=False),
    out_type=[
        jax.ShapeDtypeStruct((E, 576), jnp.float32),
        jax.ShapeDtypeStruct((E, 576), jnp.float32),
    ],
    mesh=_MESH,
    scratch_types=[
        pltpu.VMEM((RB,), jnp.int32),
        pltpu.VMEM((RB,), jnp.int32),
        pltpu.VMEM((RB, 576), jnp.float32),
        pltpu.VMEM((RB, 576), jnp.float32),
    ],
)
def _sc_grows(ha_hbm, h1_hbm, idxi_hbm, idxj_hbm, gi_hbm, gj_hbm,
              ii_v, ij_v, ri_v, rj_v):
    base = _wid() * SPAN

    def chunk(k, _):
        off = pl.multiple_of(base + k * RB, 8)
        pltpu.sync_copy(idxi_hbm.at[pl.ds(off, RB)], ii_v)
        pltpu.sync_copy(idxj_hbm.at[pl.ds(off, RB)], ij_v)
        pltpu.sync_copy(ha_hbm.at[ii_v], ri_v)
        pltpu.sync_copy(h1_hbm.at[ij_v], rj_v)
        pltpu.sync_copy(ri_v, gi_hbm.at[pl.ds(off, RB)])
        pltpu.sync_copy(rj_v, gj_hbm.at[pl.ds(off, RB)])
        return 0

    lax.fori_loop(0, SPAN // RB, chunk, 0)


@functools.partial(
    pl.kernel,
    compiler_params=pltpu.CompilerParams(use_tc_tiling_on_sc=False),
    out_type=jax.ShapeDtypeStruct((NC, 4, N, 144), jnp.float32),
    mesh=_MESH,
    scratch_types=[
        pltpu.VMEM_SHARED((N, 144), jnp.float32),
        pltpu.VMEM((RB,), jnp.int32),
        pltpu.VMEM((RB, 144), jnp.float32),
        pltpu.VMEM((25, 144), jnp.float32),
    ],
)
def _sc_srows(p_hbm, idx_hbm, out_hbm, acc, idx_v, rows_v, zb):
    cc = lax.axis_index("c")
    sid = lax.axis_index("s")
    base = _wid() * SPAN

    def zr(i, _):
        def zc(g, _):
            zb[i, pl.ds(g * 16, 16)] = jnp.zeros((16,), jnp.float32)
            return 0
        lax.fori_loop(0, 9, zc, 0)
        return 0

    lax.fori_loop(0, 25, zr, 0)

    for fc in range(4):
        r0 = sid * 625

        def zs(k, _):
            pltpu.sync_copy(zb, acc.at[pl.ds(r0 + k * 25, 25)])
            return 0

        lax.fori_loop(0, 25, zs, 0)
        plsc.subcore_barrier()

        def chunk(k, _):
            off = pl.multiple_of(base + k * RB, 8)
            pltpu.sync_copy(idx_hbm.at[pl.ds(off, RB)], idx_v)
            pltpu.sync_copy(p_hbm.at[pl.ds(off, RB), pl.ds(fc * 144, 144)], rows_v)
            pltpu.sync_copy(rows_v, acc.at[idx_v], add=True)
            return 0

        lax.fori_loop(0, SPAN // RB, chunk, 0)
        plsc.subcore_barrier()

        @pl.when(sid == 0)
        def _():
            pltpu.sync_copy(acc, out_hbm.at[cc, fc])

        plsc.subcore_barrier()


def _k1_body(disp_ref, zj_ref, et_ref, wrad_ref, wtd_ref, wb1_ref, wb2_ref,
             bda1_ref, rec_ref, sh_ref, l1_ref, mg_ref):
    pid = pl.program_id(0)
    d = disp_ref[...]
    x = d[:, 0:1]; y = d[:, 1:2]; z = d[:, 2:3]
    r2 = x * x + y * y + z * z
    r = jnp.sqrt(r2 + 1e-12)
    ux = x / r; uy = y / r; uz = z / r
    k = lax.broadcasted_iota(jnp.int32, (EB, NR), 1).astype(jnp.float32) + 1.0
    arg = jnp.pi * (k * r / CUT)
    rb0 = jnp.sin(arg) / arg
    env = jnp.where(r < CUT, 0.5 * (jnp.cos(jnp.pi * r / CUT) + 1.0), 0.0)
    rb = rb0 * env
    c0 = 0.28209479177387814
    c1c = 0.4886025119029199
    c2c = 1.0925484305920792
    sh_cols = [
        jnp.full((EB, 1), c0, jnp.float32),
        c1c * uy, c1c * uz, c1c * ux,
        c2c * ux * uy, c2c * uy * uz,
        0.31539156525252005 * (3.0 * uz * uz - 1.0),
        c2c * ux * uz,
        0.5462742152960396 * (ux * ux - uy * uy),
        jnp.zeros((EB, 7), jnp.float32),
    ]
    sh = jnp.concatenate(sh_cols, axis=1)
    zj = zj_ref[...]
    io = lax.broadcasted_iota(jnp.int32, (EB, 128), 1)
    oh = (zj == io).astype(jnp.float32)
    e_j = jnp.dot(oh, et_ref[...], preferred_element_type=jnp.float32)
    y0 = jnp.dot(rb, wrad_ref[...], preferred_element_type=jnp.float32) * e_j
    t = jnp.dot(y0, wtd_ref[...], preferred_element_type=jnp.float32)
    c1 = jnp.dot(rb, wb1_ref[...], preferred_element_type=jnp.float32)
    c2 = jnp.dot(rb, wb2_ref[...], preferred_element_type=jnp.float32)
    tc1 = t * jnp.concatenate([c1, c1, c1], axis=1)
    p = jnp.dot(tc1, bda1_ref[...], preferred_element_type=jnp.float32)
    sp = t * p
    s0 = jnp.sum(sp[:, 0:64], axis=1, keepdims=True)
    s1 = jnp.sum(sp[:, 64:128], axis=1, keepdims=True)
    s2 = jnp.sum(sp[:, 128:192], axis=1, keepdims=True)
    sh3 = sh * sh * sh
    S30 = sh3[:, 0:1]
    S31 = jnp.sum(sh3[:, 1:4], axis=1, keepdims=True)
    S32 = jnp.sum(sh3[:, 4:9], axis=1, keepdims=True)
    l1 = SCALE * (S30 * s0 + S31 * s1 + S32 * s2)
    rec_ref[...] = jnp.concatenate([t, c2], axis=1)
    sh_ref[...] = sh
    l1_ref[...] = l1
    bmax = jnp.max(l1)

    @pl.when(pid == 0)
    def _():
        mg_ref[...] = jnp.full((8, 128), NEG, jnp.float32)

    mg_ref[...] = jnp.maximum(mg_ref[...], jnp.full((8, 128), bmax))


def _k1(disp_p, zj, etp, wrad, wtd, wb1, wb2, bda1):
    grid = (E // EB,)
    return pl.pallas_call(
        _k1_body,
        grid=grid,
        in_specs=[
            pl.BlockSpec((EB, 8), lambda i: (i, 0)),
            pl.BlockSpec((EB, 1), lambda i: (i, 0)),
            pl.BlockSpec((128, 64), lambda i: (0, 0)),
            pl.BlockSpec((NR, 64), lambda i: (0, 0)),
            pl.BlockSpec((64, 192), lambda i: (0, 0)),
            pl.BlockSpec((NR, 64), lambda i: (0, 0)),
            pl.BlockSpec((NR, 64), lambda i: (0, 0)),
            pl.BlockSpec((192, 192), lambda i: (0, 0)),
        ],
        out_specs=[
            pl.BlockSpec((EB, 256), lambda i: (i, 0)),
            pl.BlockSpec((EB, 16), lambda i: (i, 0)),
            pl.BlockSpec((EB, 1), lambda i: (i, 0)),
            pl.BlockSpec((8, 128), lambda i: (0, 0)),
        ],
        out_shape=[
            jax.ShapeDtypeStruct((E, 256), jnp.float32),
            jax.ShapeDtypeStruct((E, 16), jnp.float32),
            jax.ShapeDtypeStruct((E, 1), jnp.float32),
            jax.ShapeDtypeStruct((8, 128), jnp.float32),
        ],
    )(disp_p, zj, etp, wrad, wtd, wb1, wb2, bda1)


def _k4_body(att_ref, sh_ref, rec_ref, p1_ref):
    w = att_ref[...] * sh_ref[...]
    rec = rec_ref[...]
    cols = [w[:, n:n + 1] * rec[:, DEG[n] * 64:(DEG[n] + 1) * 64] for n in range(9)]
    p1_ref[...] = jnp.concatenate(cols, axis=1)


def _k4(att1, sh, rec):
    return pl.pallas_call(
        _k4_body,
        grid=(E // EB,),
        in_specs=[
            pl.BlockSpec((EB, 1), lambda i: (i, 0)),
            pl.BlockSpec((EB, 16), lambda i: (i, 0)),
            pl.BlockSpec((EB, 256), lambda i: (i, 0)),
        ],
        out_specs=pl.BlockSpec((EB, 576), lambda i: (i, 0)),
        out_shape=jax.ShapeDtypeStruct((E, 576), jnp.float32),
    )(att1, sh, rec)


def _k6_body(msgp_ref, bd1_ref, bda2_ref, h1_ref, ha_ref):
    m = msgp_ref[...]
    msg = jnp.concatenate([m[0, q] + m[1, q] for q in range(4)], axis=1)
    h = jnp.dot(msg, bd1_ref[...], preferred_element_type=jnp.float32)
    h1_ref[...] = h
    ha_ref[...] = jnp.dot(h, bda2_ref[...], preferred_element_type=jnp.float32)


def _k6(msgp, bd1, bda2):
    return pl.pallas_call(
        _k6_body,
        grid=(N // NB,),
        in_specs=[
            pl.BlockSpec((2, 4, NB, 144), lambda i: (0, 0, i, 0)),
            pl.BlockSpec((576, 576), lambda i: (0, 0)),
            pl.BlockSpec((576, 576), lambda i: (0, 0)),
        ],
        out_specs=[
            pl.BlockSpec((NB, 576), lambda i: (i, 0)),
            pl.BlockSpec((NB, 576), lambda i: (i, 0)),
        ],
        out_shape=[
            jax.ShapeDtypeStruct((N, 576), jnp.float32),
            jax.ShapeDtypeStruct((N, 576), jnp.float32),
        ],
    )(msgp, bd1, bda2)


def _k8_body(gi_ref, gj_ref, rec_ref, sh_ref, l2_ref, mg_ref):
    pid = pl.program_id(0)
    zz = gi_ref[...] * gj_ref[...]
    c2 = rec_ref[:, 192:256]
    sh = sh_ref[...]
    acc = jnp.zeros((EB, 1), jnp.float32)
    for n in range(9):
        sn = jnp.sum(zz[:, n * 64:(n + 1) * 64] * c2, axis=1, keepdims=True)
        acc = acc + sh[:, n:n + 1] * sn
    l2 = SCALE * acc
    l2_ref[...] = l2
    bmax = jnp.max(l2)

    @pl.when(pid == 0)
    def _():
        mg_ref[...] = jnp.full((8, 128), NEG, jnp.float32)

    mg_ref[...] = jnp.maximum(mg_ref[...], jnp.full((8, 128), bmax))


def _k8(gi, gj, rec, sh):
    return pl.pallas_call(
        _k8_body,
        grid=(E // EB,),
        in_specs=[
            pl.BlockSpec((EB, 576), lambda i: (i, 0)),
            pl.BlockSpec((EB, 576), lambda i: (i, 0)),
            pl.BlockSpec((EB, 256), lambda i: (i, 0)),
            pl.BlockSpec((EB, 16), lambda i: (i, 0)),
        ],
        out_specs=[
            pl.BlockSpec((EB, 1), lambda i: (i, 0)),
            pl.BlockSpec((8, 128), lambda i: (0, 0)),
        ],
        out_shape=[
            jax.ShapeDtypeStruct((E, 1), jnp.float32),
            jax.ShapeDtypeStruct((8, 128), jnp.float32),
        ],
    )(gi, gj, rec, sh)


def _k10_body(att_ref, gj_ref, p2_ref):
    p2_ref[...] = att_ref[...] * gj_ref[...]


def _k10(att2, gj):
    return pl.pallas_call(
        _k10_body,
        grid=(E // EB,),
        in_specs=[
            pl.BlockSpec((EB, 1), lambda i: (i, 0)),
            pl.BlockSpec((EB, 576), lambda i: (i, 0)),
        ],
        out_specs=pl.BlockSpec((EB, 576), lambda i: (i, 0)),
        out_shape=jax.ShapeDtypeStruct((E, 576), jnp.float32),
    )(att2, gj)


def _k12_body(h1_ref, msgp_ref, bd2_ref, an_ref, et_ref, we_ref, be_ref, out_ref):
    m = msgp_ref[...]
    msg = jnp.concatenate([m[0, q] + m[1, q] for q in range(4)], axis=1)
    out = h1_ref[...] + jnp.dot(msg, bd2_ref[...], preferred_element_type=jnp.float32)
    an = an_ref[...]
    io = lax.broadcasted_iota(jnp.int32, (NB, 128), 1)
    oh = (an == io).astype(jnp.float32)
    en = jnp.dot(oh, et_ref[...], preferred_element_type=jnp.float32)
    emb = jnp.dot(en, we_ref[...], preferred_element_type=jnp.float32) + be_ref[0:1, :]
    out_ref[...] = jnp.concatenate([out[:, 0:64] + emb, out[:, 64:576]], axis=1)


def _k12(h1, msgp2, bd2, an2, etp, we, be8):
    return pl.pallas_call(
        _k12_body,
        grid=(N // NB,),
        in_specs=[
            pl.BlockSpec((NB, 576), lambda i: (i, 0)),
            pl.BlockSpec((2, 4, NB, 144), lambda i: (0, 0, i, 0)),
            pl.BlockSpec((576, 576), lambda i: (0, 0)),
            pl.BlockSpec((NB, 1), lambda i: (i, 0)),
            pl.BlockSpec((128, 64), lambda i: (0, 0)),
            pl.BlockSpec((64, 64), lambda i: (0, 0)),
            pl.BlockSpec((8, 64), lambda i: (0, 0)),
        ],
        out_specs=pl.BlockSpec((NB, 576), lambda i: (i, 0)),
        out_shape=jax.ShapeDtypeStruct((N, 576), jnp.float32),
    )(h1, msgp2, bd2, an2, etp, we, be8)


def kernel(atomic_numbers, neighbour_indices, neighbour_displacements,
           embed_table, W_rad, W_td,
           Wq1, Wk1, Wv1, Wo1, Wb1,
           Wq2, Wk2, Wv2, Wo2, Wb2,
           W_embed, b_embed):
    an = atomic_numbers.astype(jnp.int32)
    idx_i = neighbour_indices[:, 0].astype(jnp.int32)
    idx_j = neighbour_indices[:, 1].astype(jnp.int32)
    disp_p = jnp.pad(neighbour_displacements, ((0, 0), (0, 5)))
    etp = jnp.pad(embed_table, ((0, 28), (0, 0)))
    wtd = jnp.transpose(W_td, (1, 0, 2)).reshape(64, 192)
    A1 = Wq1 @ Wk1.T
    A2 = Wq2 @ Wk2.T
    bda1 = jnp.kron(jnp.eye(3, dtype=jnp.float32), A1.T)
    bd1 = jnp.kron(jnp.eye(9, dtype=jnp.float32), Wv1 @ Wo1)
    bda2 = jnp.kron(jnp.eye(9, dtype=jnp.float32), A2)
    bd2 = jnp.kron(jnp.eye(9, dtype=jnp.float32), Wv2 @ Wo2)
    be8 = jnp.broadcast_to(b_embed[None, :], (8, 64))
    an2 = an[:, None]

    zj = _sc_zj(an, idx_j)[:, None]

    rec, sh, l1, mg_t = _k1(disp_p, zj, etp, W_rad, wtd, Wb1, Wb2, bda1)
    mg16 = jnp.broadcast_to(mg_t[0:1, 0], (16,))
    l1f = l1.reshape(E)

    s1p = _sc_ssum(l1f, idx_i, mg16)
    att1 = _sc_att(s1p[0], s1p[1], l1f, idx_i, mg16)[:, None]

    p1 = _k4(att1, sh, rec)
    msgp1 = _sc_srows(p1, idx_i)
    h1, ha = _k6(msgp1, bd1, bda2)

    gi, gj = _sc_grows(ha, h1, idx_i, idx_j)

    l2, mg2_t = _k8(gi, gj, rec, sh)
    mg216 = jnp.broadcast_to(mg2_t[0:1, 0], (16,))
    l2f = l2.reshape(E)

    s2p = _sc_ssum(l2f, idx_i, mg216)
    att2 = _sc_att(s2p[0], s2p[1], l2f, idx_i, mg216)[:, None]

    p2 = _k10(att2, gj)
    msgp2 = _sc_srows(p2, idx_i)

    out = _k12(h1, msgp2, bd2, an2, etp, W_embed, be8)
    return out.reshape(N, 9, F)


# speedup vs baseline: 7.0650x; 1.1073x over previous
import functools
import numpy as np
import jax
import jax.numpy as jnp
from jax import lax
from jax.experimental import pallas as pl
from jax.experimental.pallas import tpu as pltpu
from jax.experimental.pallas import tpu_sc as plsc

N = 10000
E = 160000
F = 64
NR = 16
CUT = 5.0
DEG = (0, 1, 1, 1, 2, 2, 2, 2, 2)
SCALE = 1.0 / np.sqrt(9 * F)
EB = 3200
NB = 1000
NEG = -1e30

NC = 2
NS = 16
NW = NC * NS
SPAN = E // NW
CH = 1000
NV = CH // 16 + 1
RB = 40

_MESH = plsc.VectorSubcoreMesh(core_axis_name="c", subcore_axis_name="s",
                               num_cores=NC, num_subcores=NS)


def _wid():
    return lax.axis_index("s") * NC + lax.axis_index("c")


CHK = 128
NCHUNK = E // CHK


def _nck(w):
    return jnp.where(w < NCHUNK - 39 * NW, 40, 39)


@functools.partial(
    pl.kernel,
    compiler_params=pltpu.CompilerParams(use_tc_tiling_on_sc=False),
    out_type=jax.ShapeDtypeStruct((E,), jnp.int32),
    mesh=_MESH,
    scratch_types=[
        pltpu.VMEM((CHK,), jnp.int32),
        pltpu.VMEM((CHK,), jnp.int32),
    ],
)
def _sc_zj(an_hbm, idxj_hbm, out_hbm, idx_v, out_v):
    w = _wid()

    def chunk(k, _):
        off = pl.multiple_of((w + NW * k) * CHK, CHK)
        pltpu.sync_copy(idxj_hbm.at[pl.ds(off, CHK)], idx_v)
        pltpu.sync_copy(an_hbm.at[idx_v], out_v)
        pltpu.sync_copy(out_v, out_hbm.at[pl.ds(off, CHK)])
        return 0

    lax.fori_loop(0, _nck(w), chunk, 0)


@functools.partial(
    pl.kernel,
    compiler_params=pltpu.CompilerParams(use_tc_tiling_on_sc=False),
    out_type=jax.ShapeDtypeStruct((NC, N), jnp.float32),
    mesh=_MESH,
    scratch_types=[
        pltpu.VMEM_SHARED((N,), jnp.float32),
        pltpu.VMEM((1024,), jnp.float32),
        pltpu.VMEM((CHK,), jnp.float32),
        pltpu.VMEM((CHK,), jnp.int32),
        pltpu.VMEM((CHK,), jnp.float32),
        pltpu.VMEM((16,), jnp.float32),
    ],
)
def _sc_ssum(l_hbm, idx_hbm, mg_hbm, out_hbm, acc, zb, l_v, idx_v, ex_v, mg_v):
    cc = lax.axis_index("c")
    sid = lax.axis_index("s")
    w = _wid()
    pltpu.sync_copy(mg_hbm, mg_v)

    @pl.when(sid == 0)
    def _():
        def zv(j, _):
            zb[pl.ds(j * 16, 16)] = jnp.zeros((16,), jnp.float32)
            return 0
        lax.fori_loop(0, 64, zv, 0)

        def zc(k, _):
            pltpu.sync_copy(zb, acc.at[pl.ds(pl.multiple_of(k * 1024, 1024), 1024)])
            return 0
        lax.fori_loop(0, N // 1024, zc, 0)
        pltpu.sync_copy(zb.at[pl.ds(0, N - 1024 * (N // 1024))],
                        acc.at[pl.ds(1024 * (N // 1024), N - 1024 * (N // 1024))])

    plsc.subcore_barrier()
    mg = mg_v[...]

    def chunk(k, _):
        off = pl.multiple_of((w + NW * k) * CHK, CHK)
        pltpu.sync_copy(l_hbm.at[pl.ds(off, CHK)], l_v)
        pltpu.sync_copy(idx_hbm.at[pl.ds(off, CHK)], idx_v)

        def vloop(j, _):
            ex_v[pl.ds(j * 16, 16)] = jnp.exp(l_v[pl.ds(j * 16, 16)] - mg)
            return 0

        lax.fori_loop(0, CHK // 16, vloop, 0)
        pltpu.sync_copy(ex_v, acc.at[idx_v], add=True)
        return 0

    lax.fori_loop(0, _nck(w), chunk, 0)
    plsc.subcore_barrier()

    @pl.when(sid == 0)
    def _():
        pltpu.sync_copy(acc, out_hbm.at[cc])


@functools.partial(
    pl.kernel,
    compiler_params=pltpu.CompilerParams(use_tc_tiling_on_sc=False),
    out_type=jax.ShapeDtypeStruct((E,), jnp.float32),
    mesh=_MESH,
    scratch_types=[
        pltpu.VMEM((CHK,), jnp.float32),
        pltpu.VMEM((CHK,), jnp.int32),
        pltpu.VMEM((CHK,), jnp.float32),
        pltpu.VMEM((CHK,), jnp.float32),
        pltpu.VMEM((CHK,), jnp.float32),
        pltpu.VMEM((16,), jnp.float32),
    ],
)
def _sc_att(s0_hbm, s1_hbm, l_hbm, idx_hbm, mg_hbm, out_hbm,
            l_v, idx_v, s0_v, s1_v, o_v, mg_v):
    w = _wid()
    pltpu.sync_copy(mg_hbm, mg_v)
    mg = mg_v[...]

    def chunk(k, _):
        off = pl.multiple_of((w + NW * k) * CHK, CHK)
        pltpu.sync_copy(l_hbm.at[pl.ds(off, CHK)], l_v)
        pltpu.sync_copy(idx_hbm.at[pl.ds(off, CHK)], idx_v)
        pltpu.sync_copy(s0_hbm.at[idx_v], s0_v)
        pltpu.sync_copy(s1_hbm.at[idx_v], s1_v)

        def vloop(j, _):
            sl = pl.ds(j * 16, 16)
            ex = jnp.exp(l_v[sl] - mg)
            o_v[sl] = ex / (s0_v[sl] + s1_v[sl] + 1e-9)
            return 0

        lax.fori_loop(0, CHK // 16, vloop, 0)
        pltpu.sync_copy(o_v, out_hbm.at[pl.ds(off, CHK)])
        return 0

    lax.fori_loop(0, _nck(w), chunk, 0)


@functools.partial(
    pl.kernel,
    compiler_params=pltpu.CompilerParams(use_tc_tiling_on_sc=False),
    out_type=[
        jax.ShapeDtypeStruct((E, 576), jnp.float32),
        jax.ShapeDtypeStruct((E, 576), jnp.float32),
    ],
    mesh=_MESH,
    scratch_types=[
        pltpu.VMEM((RB,), jnp.int32),
        pltpu.VMEM((RB,), jnp.int32),
        pltpu.VMEM((RB, 576), jnp.float32),
        pltpu.VMEM((RB, 576), jnp.float32),
    ],
)
def _sc_grows(ha_hbm, h1_hbm, idxi_hbm, idxj_hbm, gi_hbm, gj_hbm,
              ii_v, ij_v, ri_v, rj_v):
    base = _wid() * SPAN

    def chunk(k, _):
        off = pl.multiple_of(base + k * RB, 8)
        pltpu.sync_copy(idxi_hbm.at[pl.ds(off, RB)], ii_v)
        pltpu.sync_copy(idxj_hbm.at[pl.ds(off, RB)], ij_v)
        pltpu.sync_copy(ha_hbm.at[ii_v], ri_v)
        pltpu.sync_copy(h1_hbm.at[ij_v], rj_v)
        pltpu.sync_copy(ri_v, gi_hbm.at[pl.ds(off, RB)])
        pltpu.sync_copy(rj_v, gj_hbm.at[pl.ds(off, RB)])
        return 0

    lax.fori_loop(0, SPAN // RB, chunk, 0)


@functools.partial(
    pl.kernel,
    compiler_params=pltpu.CompilerParams(use_tc_tiling_on_sc=False),
    out_type=jax.ShapeDtypeStruct((NC, 4, N, 144), jnp.float32),
    mesh=_MESH,
    scratch_types=[
        pltpu.VMEM_SHARED((N, 144), jnp.float32),
        pltpu.VMEM((120,), jnp.int32),
        pltpu.VMEM((120, 144), jnp.float32),
        pltpu.VMEM((80,), jnp.int32),
        pltpu.VMEM((80, 144), jnp.float32),
        pltpu.VMEM((25, 144), jnp.float32),
    ],
)
def _sc_srows(p_hbm, idx_hbm, out_hbm, acc, idx_a, rows_a, idx_b, rows_b, zb):
    cc = lax.axis_index("c")
    sid = lax.axis_index("s")
    base = _wid() * SPAN

    def zr(i, _):
        def zc(g, _):
            zb[i, pl.ds(g * 16, 16)] = jnp.zeros((16,), jnp.float32)
            return 0
        lax.fori_loop(0, 9, zc, 0)
        return 0

    lax.fori_loop(0, 25, zr, 0)

    for fc in range(4):
        r0 = sid * 625

        def zs(k, _):
            pltpu.sync_copy(zb, acc.at[pl.ds(r0 + k * 25, 25)])
            return 0

        lax.fori_loop(0, 25, zs, 0)
        plsc.subcore_barrier()

        def chunk(k, _):
            off = pl.multiple_of(base + k * 120, 8)
            pltpu.sync_copy(idx_hbm.at[pl.ds(off, 120)], idx_a)
            pltpu.sync_copy(p_hbm.at[pl.ds(off, 120), pl.ds(fc * 144, 144)], rows_a)
            pltpu.sync_copy(rows_a, acc.at[idx_a], add=True)
            return 0

        lax.fori_loop(0, 41, chunk, 0)
        offt = pl.multiple_of(base + 4920, 8)
        pltpu.sync_copy(idx_hbm.at[pl.ds(offt, 80)], idx_b)
        pltpu.sync_copy(p_hbm.at[pl.ds(offt, 80), pl.ds(fc * 144, 144)], rows_b)
        pltpu.sync_copy(rows_b, acc.at[idx_b], add=True)
        plsc.subcore_barrier()

        @pl.when(sid == 0)
        def _():
            pltpu.sync_copy(acc, out_hbm.at[cc, fc])

        plsc.subcore_barrier()


def _k1_body(disp_ref, zj_ref, et_ref, wrad_ref, wtd_ref, wb1_ref, wb2_ref,
             bda1_ref, rec_ref, sh_ref, l1_ref, mg_ref):
    pid = pl.program_id(0)
    d = disp_ref[...]
    x = d[:, 0:1]; y = d[:, 1:2]; z = d[:, 2:3]
    r2 = x * x + y * y + z * z
    r = jnp.sqrt(r2 + 1e-12)
    ux = x / r; uy = y / r; uz = z / r
    k = lax.broadcasted_iota(jnp.int32, (EB, NR), 1).astype(jnp.float32) + 1.0
    arg = jnp.pi * (k * r / CUT)
    rb0 = jnp.sin(arg) / arg
    env = jnp.where(r < CUT, 0.5 * (jnp.cos(jnp.pi * r / CUT) + 1.0), 0.0)
    rb = rb0 * env
    c0 = 0.28209479177387814
    c1c = 0.4886025119029199
    c2c = 1.0925484305920792
    sh_cols = [
        jnp.full((EB, 1), c0, jnp.float32),
        c1c * uy, c1c * uz, c1c * ux,
        c2c * ux * uy, c2c * uy * uz,
        0.31539156525252005 * (3.0 * uz * uz - 1.0),
        c2c * ux * uz,
        0.5462742152960396 * (ux * ux - uy * uy),
        jnp.zeros((EB, 7), jnp.float32),
    ]
    sh = jnp.concatenate(sh_cols, axis=1)
    zj = zj_ref[...]
    io = lax.broadcasted_iota(jnp.int32, (EB, 128), 1)
    oh = (zj == io).astype(jnp.float32)
    e_j = jnp.dot(oh, et_ref[...], preferred_element_type=jnp.float32)
    y0 = jnp.dot(rb, wrad_ref[...], preferred_element_type=jnp.float32) * e_j
    t = jnp.dot(y0, wtd_ref[...], preferred_element_type=jnp.float32)
    c1 = jnp.dot(rb, wb1_ref[...], preferred_element_type=jnp.float32)
    c2 = jnp.dot(rb, wb2_ref[...], preferred_element_type=jnp.float32)
    tc1 = t * jnp.concatenate([c1, c1, c1], axis=1)
    p = jnp.dot(tc1, bda1_ref[...], preferred_element_type=jnp.float32)
    sp = t * p
    s0 = jnp.sum(sp[:, 0:64], axis=1, keepdims=True)
    s1 = jnp.sum(sp[:, 64:128], axis=1, keepdims=True)
    s2 = jnp.sum(sp[:, 128:192], axis=1, keepdims=True)
    sh3 = sh * sh * sh
    S30 = sh3[:, 0:1]
    S31 = jnp.sum(sh3[:, 1:4], axis=1, keepdims=True)
    S32 = jnp.sum(sh3[:, 4:9], axis=1, keepdims=True)
    l1 = SCALE * (S30 * s0 + S31 * s1 + S32 * s2)
    rec_ref[...] = jnp.concatenate([t, c2], axis=1)
    sh_ref[...] = sh
    l1_ref[...] = l1
    bmax = jnp.max(l1)

    @pl.when(pid == 0)
    def _():
        mg_ref[...] = jnp.full((8, 128), NEG, jnp.float32)

    mg_ref[...] = jnp.maximum(mg_ref[...], jnp.full((8, 128), bmax))


def _k1(disp_p, zj, etp, wrad, wtd, wb1, wb2, bda1):
    grid = (E // EB,)
    return pl.pallas_call(
        _k1_body,
        grid=grid,
        in_specs=[
            pl.BlockSpec((EB, 8), lambda i: (i, 0)),
            pl.BlockSpec((EB, 1), lambda i: (i, 0)),
            pl.BlockSpec((128, 64), lambda i: (0, 0)),
            pl.BlockSpec((NR, 64), lambda i: (0, 0)),
            pl.BlockSpec((64, 192), lambda i: (0, 0)),
            pl.BlockSpec((NR, 64), lambda i: (0, 0)),
            pl.BlockSpec((NR, 64), lambda i: (0, 0)),
            pl.BlockSpec((192, 192), lambda i: (0, 0)),
        ],
        out_specs=[
            pl.BlockSpec((EB, 256), lambda i: (i, 0)),
            pl.BlockSpec((EB, 16), lambda i: (i, 0)),
            pl.BlockSpec((EB, 1), lambda i: (i, 0)),
            pl.BlockSpec((8, 128), lambda i: (0, 0)),
        ],
        out_shape=[
            jax.ShapeDtypeStruct((E, 256), jnp.float32),
            jax.ShapeDtypeStruct((E, 16), jnp.float32),
            jax.ShapeDtypeStruct((E, 1), jnp.float32),
            jax.ShapeDtypeStruct((8, 128), jnp.float32),
        ],
    )(disp_p, zj, etp, wrad, wtd, wb1, wb2, bda1)


def _k4_body(att_ref, sh_ref, rec_ref, p1_ref):
    w = att_ref[...] * sh_ref[...]
    rec = rec_ref[...]
    cols = [w[:, n:n + 1] * rec[:, DEG[n] * 64:(DEG[n] + 1) * 64] for n in range(9)]
    p1_ref[...] = jnp.concatenate(cols, axis=1)


def _k4(att1, sh, rec):
    return pl.pallas_call(
        _k4_body,
        grid=(E // EB,),
        in_specs=[
            pl.BlockSpec((EB, 1), lambda i: (i, 0)),
            pl.BlockSpec((EB, 16), lambda i: (i, 0)),
            pl.BlockSpec((EB, 256), lambda i: (i, 0)),
        ],
        out_specs=pl.BlockSpec((EB, 576), lambda i: (i, 0)),
        out_shape=jax.ShapeDtypeStruct((E, 576), jnp.float32),
    )(att1, sh, rec)


def _k6_body(msgp_ref, bd1_ref, bda2_ref, h1_ref, ha_ref):
    m = msgp_ref[...]
    msg = jnp.concatenate([m[0, q] + m[1, q] for q in range(4)], axis=1)
    h = jnp.dot(msg, bd1_ref[...], preferred_element_type=jnp.float32)
    h1_ref[...] = h
    ha_ref[...] = jnp.dot(h, bda2_ref[...], preferred_element_type=jnp.float32)


def _k6(msgp, bd1, bda2):
    return pl.pallas_call(
        _k6_body,
        grid=(N // NB,),
        in_specs=[
            pl.BlockSpec((2, 4, NB, 144), lambda i: (0, 0, i, 0)),
            pl.BlockSpec((576, 576), lambda i: (0, 0)),
            pl.BlockSpec((576, 576), lambda i: (0, 0)),
        ],
        out_specs=[
            pl.BlockSpec((NB, 576), lambda i: (i, 0)),
            pl.BlockSpec((NB, 576), lambda i: (i, 0)),
        ],
        out_shape=[
            jax.ShapeDtypeStruct((N, 576), jnp.float32),
            jax.ShapeDtypeStruct((N, 576), jnp.float32),
        ],
    )(msgp, bd1, bda2)


def _k8_body(gi_ref, gj_ref, rec_ref, sh_ref, l2_ref, mg_ref):
    pid = pl.program_id(0)
    zz = gi_ref[...] * gj_ref[...]
    c2 = rec_ref[:, 192:256]
    sh = sh_ref[...]
    acc = jnp.zeros((EB, 1), jnp.float32)
    for n in range(9):
        sn = jnp.sum(zz[:, n * 64:(n + 1) * 64] * c2, axis=1, keepdims=True)
        acc = acc + sh[:, n:n + 1] * sn
    l2 = SCALE * acc
    l2_ref[...] = l2
    bmax = jnp.max(l2)

    @pl.when(pid == 0)
    def _():
        mg_ref[...] = jnp.full((8, 128), NEG, jnp.float32)

    mg_ref[...] = jnp.maximum(mg_ref[...], jnp.full((8, 128), bmax))


def _k8(gi, gj, rec, sh):
    return pl.pallas_call(
        _k8_body,
        grid=(E // EB,),
        in_specs=[
            pl.BlockSpec((EB, 576), lambda i: (i, 0)),
            pl.BlockSpec((EB, 576), lambda i: (i, 0)),
            pl.BlockSpec((EB, 256), lambda i: (i, 0)),
            pl.BlockSpec((EB, 16), lambda i: (i, 0)),
        ],
        out_specs=[
            pl.BlockSpec((EB, 1), lambda i: (i, 0)),
            pl.BlockSpec((8, 128), lambda i: (0, 0)),
        ],
        out_shape=[
            jax.ShapeDtypeStruct((E, 1), jnp.float32),
            jax.ShapeDtypeStruct((8, 128), jnp.float32),
        ],
    )(gi, gj, rec, sh)


def _k10_body(att_ref, gj_ref, p2_ref):
    p2_ref[...] = att_ref[...] * gj_ref[...]


def _k10(att2, gj):
    return pl.pallas_call(
        _k10_body,
        grid=(E // EB,),
        in_specs=[
            pl.BlockSpec((EB, 1), lambda i: (i, 0)),
            pl.BlockSpec((EB, 576), lambda i: (i, 0)),
        ],
        out_specs=pl.BlockSpec((EB, 576), lambda i: (i, 0)),
        out_shape=jax.ShapeDtypeStruct((E, 576), jnp.float32),
    )(att2, gj)


def _k12_body(h1_ref, msgp_ref, bd2_ref, an_ref, et_ref, we_ref, be_ref, out_ref):
    m = msgp_ref[...]
    msg = jnp.concatenate([m[0, q] + m[1, q] for q in range(4)], axis=1)
    out = h1_ref[...] + jnp.dot(msg, bd2_ref[...], preferred_element_type=jnp.float32)
    an = an_ref[...]
    io = lax.broadcasted_iota(jnp.int32, (NB, 128), 1)
    oh = (an == io).astype(jnp.float32)
    en = jnp.dot(oh, et_ref[...], preferred_element_type=jnp.float32)
    emb = jnp.dot(en, we_ref[...], preferred_element_type=jnp.float32) + be_ref[0:1, :]
    out_ref[...] = jnp.concatenate([out[:, 0:64] + emb, out[:, 64:576]], axis=1)


def _k12(h1, msgp2, bd2, an2, etp, we, be8):
    return pl.pallas_call(
        _k12_body,
        grid=(N // NB,),
        in_specs=[
            pl.BlockSpec((NB, 576), lambda i: (i, 0)),
            pl.BlockSpec((2, 4, NB, 144), lambda i: (0, 0, i, 0)),
            pl.BlockSpec((576, 576), lambda i: (0, 0)),
            pl.BlockSpec((NB, 1), lambda i: (i, 0)),
            pl.BlockSpec((128, 64), lambda i: (0, 0)),
            pl.BlockSpec((64, 64), lambda i: (0, 0)),
            pl.BlockSpec((8, 64), lambda i: (0, 0)),
        ],
        out_specs=pl.BlockSpec((NB, 576), lambda i: (i, 0)),
        out_shape=jax.ShapeDtypeStruct((N, 576), jnp.float32),
    )(h1, msgp2, bd2, an2, etp, we, be8)


def kernel(atomic_numbers, neighbour_indices, neighbour_displacements,
           embed_table, W_rad, W_td,
           Wq1, Wk1, Wv1, Wo1, Wb1,
           Wq2, Wk2, Wv2, Wo2, Wb2,
           W_embed, b_embed):
    an = atomic_numbers.astype(jnp.int32)
    idx_i = neighbour_indices[:, 0].astype(jnp.int32)
    idx_j = neighbour_indices[:, 1].astype(jnp.int32)
    disp_p = jnp.pad(neighbour_displacements, ((0, 0), (0, 5)))
    etp = jnp.pad(embed_table, ((0, 28), (0, 0)))
    wtd = jnp.transpose(W_td, (1, 0, 2)).reshape(64, 192)
    A1 = Wq1 @ Wk1.T
    A2 = Wq2 @ Wk2.T
    bda1 = jnp.kron(jnp.eye(3, dtype=jnp.float32), A1.T)
    bd1 = jnp.kron(jnp.eye(9, dtype=jnp.float32), Wv1 @ Wo1)
    bda2 = jnp.kron(jnp.eye(9, dtype=jnp.float32), A2)
    bd2 = jnp.kron(jnp.eye(9, dtype=jnp.float32), Wv2 @ Wo2)
    be8 = jnp.broadcast_to(b_embed[None, :], (8, 64))
    an2 = an[:, None]

    zj = _sc_zj(an, idx_j)[:, None]

    rec, sh, l1, mg_t = _k1(disp_p, zj, etp, W_rad, wtd, Wb1, Wb2, bda1)
    mg16 = jnp.broadcast_to(mg_t[0:1, 0], (16,))
    l1f = l1.reshape(E)

    s1p = _sc_ssum(l1f, idx_i, mg16)
    att1 = _sc_att(s1p[0], s1p[1], l1f, idx_i, mg16)[:, None]

    p1 = _k4(att1, sh, rec)
    msgp1 = _sc_srows(p1, idx_i)
    h1, ha = _k6(msgp1, bd1, bda2)

    gi, gj = _sc_grows(ha, h1, idx_i, idx_j)

    l2, mg2_t = _k8(gi, gj, rec, sh)
    mg216 = jnp.broadcast_to(mg2_t[0:1, 0], (16,))
    l2f = l2.reshape(E)

    s2p = _sc_ssum(l2f, idx_i, mg216)
    att2 = _sc_att(s2p[0], s2p[1], l2f, idx_i, mg216)[:, None]

    p2 = _k10(att2, gj)
    msgp2 = _sc_srows(p2, idx_i)

    out = _k12(h1, msgp2, bd2, an2, etp, W_embed, be8)
    return out.reshape(N, 9, F)
